# Initial kernel scaffold; baseline (speedup 1.0000x reference)
import functools

import jax
import jax.numpy as jnp
from jax import lax
from jax.experimental import pallas as pl
from jax.experimental.pallas import tpu as pltpu, tpu_sc as plsc

VOCAB = 100000
N_EMBD = 128
B, T = 1024, 200

NC, NS, L = 2, 16, 16
NW = NC * NS
ROWS = B * T
ROWS_W = ROWS // NW
SEQ_W = ROWS_W // T
HALF = T // 2
NV = N_EMBD // L

_EPS = 1e-5
_RSQRT_MAGIC = 0x5F3759DF


def _rsqrt16(x):
    i = plsc.bitcast(x, jnp.int32)
    y = plsc.bitcast(jnp.int32(_RSQRT_MAGIC) - (i >> 1), jnp.float32)
    for _ in range(3):
        y = y * (1.5 - 0.5 * x * y * y)
    return y


def _sc_body(ids_hbm, tt_hbm, tok_hbm, pos_hbm, seg_hbm, gam_hbm, bet_hbm,
             out_hbm, idx_v, ttv_v, buf_v, pos_v, seg_v, gb_v, gsem):
    wid = lax.axis_index("s") * NC + lax.axis_index("c")

    pltpu.sync_copy(ids_hbm.at[pl.ds(wid * (2 * SEQ_W), 2 * SEQ_W)], idx_v)
    pltpu.sync_copy(tt_hbm.at[pl.ds(wid * (2 * SEQ_W), 2 * SEQ_W)], ttv_v)
    pltpu.sync_copy(pos_hbm.at[pl.ds(0, T)], pos_v)
    pltpu.sync_copy(seg_hbm, seg_v)
    pltpu.sync_copy(gam_hbm, gb_v.at[0])
    pltpu.sync_copy(bet_hbm, gb_v.at[1])

    seg0 = [seg_v[0, pl.ds(L * k, L)] for k in range(NV)]
    dseg = [seg_v[1, pl.ds(L * k, L)] - seg0[k] for k in range(NV)]
    gam = [gb_v[0, pl.ds(L * k, L)] for k in range(NV)]
    bet = [gb_v[1, pl.ds(L * k, L)] for k in range(NV)]

    def do_seq(c, _):
        cp0 = pltpu.async_copy(tok_hbm.at[idx_v.at[2 * c]],
                               buf_v.at[pl.ds(0, HALF)], gsem)
        cp1 = pltpu.async_copy(tok_hbm.at[idx_v.at[2 * c + 1]],
                               buf_v.at[pl.ds(HALF, HALF)], gsem)
        cp0.wait()
        cp1.wait()

        def do_row(j, _):
            half = j // HALF
            col = j - half * HALF
            ttb = plsc.load_gather(
                ttv_v, [jnp.full((L,), 2 * c, jnp.int32) + half,
                        jnp.full((L,), 0, jnp.int32) + col])
            ttf = ttb.astype(jnp.float32)
            acc = []
            s = None
            for k in range(NV):
                a = (buf_v[j, pl.ds(L * k, L)] + pos_v[j, pl.ds(L * k, L)]
                     + (seg0[k] + ttf * dseg[k]))
                acc.append(a)
                s = a if s is None else s + a
            mean = jnp.sum(s) * (1.0 / N_EMBD)
            sq = None
            for k in range(NV):
                acc[k] = acc[k] - mean
                d2 = acc[k] * acc[k]
                sq = d2 if sq is None else sq + d2
            var = jnp.sum(sq) * (1.0 / N_EMBD)
            rstd = _rsqrt16(jnp.full((L,), var + _EPS, jnp.float32))
            for k in range(NV):
                buf_v[j, pl.ds(L * k, L)] = acc[k] * (rstd * gam[k]) + bet[k]
            return 0

        lax.fori_loop(0, T, do_row, 0)
        pltpu.sync_copy(buf_v, out_hbm.at[pl.ds((wid * SEQ_W + c) * T, T)])
        return 0

    lax.fori_loop(0, SEQ_W, do_seq, 0)


@jax.jit
def _bert_embed_sc(ids2, tt2, tok_table, pos_table, seg_table, gamma, beta):
    kern = pl.kernel(
        _sc_body,
        out_type=jax.ShapeDtypeStruct((ROWS, N_EMBD), jnp.float32),
        mesh=plsc.VectorSubcoreMesh(core_axis_name="c", subcore_axis_name="s"),
        scratch_types=[
            pltpu.VMEM((2 * SEQ_W, HALF), jnp.int32),
            pltpu.VMEM((2 * SEQ_W, HALF), jnp.int32),
            pltpu.VMEM((T, N_EMBD), jnp.float32),
            pltpu.VMEM((T, N_EMBD), jnp.float32),
            pltpu.VMEM((2, N_EMBD), jnp.float32),
            pltpu.VMEM((2, N_EMBD), jnp.float32),
            pltpu.SemaphoreType.DMA,
        ],
    )
    return kern(ids2, tt2, tok_table, pos_table, seg_table, gamma, beta)


def kernel(input_ids, token_type_ids, tok_table, pos_table, seg_table,
           ln_gamma, ln_beta):
    ids2 = jnp.asarray(input_ids, jnp.int32).reshape(ROWS // HALF, HALF)
    tt2 = jnp.asarray(token_type_ids, jnp.int32).reshape(ROWS // HALF, HALF)
    out = _bert_embed_sc(ids2, tt2, tok_table, pos_table, seg_table,
                         jnp.asarray(ln_gamma, jnp.float32),
                         jnp.asarray(ln_beta, jnp.float32))
    return out.reshape(B, T, N_EMBD)


# speedup vs baseline: 3.2292x; 3.2292x over previous
import functools

import jax
import jax.numpy as jnp
from jax import lax
from jax.experimental import pallas as pl
from jax.experimental.pallas import tpu as pltpu, tpu_sc as plsc

VOCAB = 100000
N_EMBD = 128
B, T = 1024, 200

NC, NS, L = 2, 16, 16
NW = NC * NS
ROWS = B * T
ROWS_W = ROWS // NW
SEQ_W = ROWS_W // T
HALF = T // 2
NV = N_EMBD // L

_EPS = 1e-5
_RSQRT_MAGIC = 0x5F3759DF


def _rsqrt16(x):
    i = plsc.bitcast(x, jnp.int32)
    y = plsc.bitcast(jnp.int32(_RSQRT_MAGIC) - (i >> 1), jnp.float32)
    for _ in range(3):
        y = y * (1.5 - 0.5 * x * y * y)
    return y


def _sc_body(ids_hbm, tt_hbm, tok_hbm, pos_hbm, seg_hbm, gam_hbm, bet_hbm,
             out_hbm, idx_v, ttv_v, buf_v, pos_v, seg_v, gb_v, gsem):
    wid = lax.axis_index("s") * NC + lax.axis_index("c")

    pltpu.sync_copy(ids_hbm.at[pl.ds(wid * (2 * SEQ_W), 2 * SEQ_W)], idx_v)
    pltpu.sync_copy(tt_hbm.at[pl.ds(wid * ROWS_W, ROWS_W)],
                    ttv_v.at[pl.ds(0, ROWS_W)])
    pltpu.sync_copy(pos_hbm.at[pl.ds(0, T)], pos_v)
    pltpu.sync_copy(seg_hbm, seg_v)
    pltpu.sync_copy(gam_hbm, gb_v.at[0])
    pltpu.sync_copy(bet_hbm, gb_v.at[1])

    seg0 = [seg_v[0, pl.ds(L * k, L)] for k in range(NV)]
    dseg = [seg_v[1, pl.ds(L * k, L)] - seg0[k] for k in range(NV)]
    gam = [gb_v[0, pl.ds(L * k, L)] for k in range(NV)]
    bet = [gb_v[1, pl.ds(L * k, L)] for k in range(NV)]

    def do_seq(c, _):
        cp0 = pltpu.async_copy(tok_hbm.at[idx_v.at[2 * c]],
                               buf_v.at[pl.ds(0, HALF)], gsem)
        cp1 = pltpu.async_copy(tok_hbm.at[idx_v.at[2 * c + 1]],
                               buf_v.at[pl.ds(HALF, HALF)], gsem)
        cp0.wait()
        cp1.wait()

        def do_row(j, _):
            ttf = ttv_v[pl.ds(c * T + j, L)][0].astype(jnp.float32)
            acc = []
            s = None
            for k in range(NV):
                a = (buf_v[j, pl.ds(L * k, L)] + pos_v[j, pl.ds(L * k, L)]
                     + (seg0[k] + ttf * dseg[k]))
                acc.append(a)
                s = a if s is None else s + a
            mean = jnp.sum(s) * (1.0 / N_EMBD)
            sq = None
            for k in range(NV):
                acc[k] = acc[k] - mean
                d2 = acc[k] * acc[k]
                sq = d2 if sq is None else sq + d2
            var = jnp.sum(sq) * (1.0 / N_EMBD)
            rstd = _rsqrt16(jnp.full((L,), var + _EPS, jnp.float32))
            for k in range(NV):
                buf_v[j, pl.ds(L * k, L)] = acc[k] * (rstd * gam[k]) + bet[k]
            return 0

        lax.fori_loop(0, T, do_row, 0)
        pltpu.sync_copy(buf_v, out_hbm.at[pl.ds((wid * SEQ_W + c) * T, T)])
        return 0

    lax.fori_loop(0, SEQ_W, do_seq, 0)


@jax.jit
def _bert_embed_sc(ids2, tt2, tok_table, pos_table, seg_table, gamma, beta):
    kern = pl.kernel(
        _sc_body,
        out_type=jax.ShapeDtypeStruct((ROWS, N_EMBD), jnp.float32),
        mesh=plsc.VectorSubcoreMesh(core_axis_name="c", subcore_axis_name="s"),
        compiler_params=pltpu.CompilerParams(needs_layout_passes=False),
        scratch_types=[
            pltpu.VMEM((2 * SEQ_W, HALF), jnp.int32),
            pltpu.VMEM((ROWS_W + L,), jnp.int32),
            pltpu.VMEM((T, N_EMBD), jnp.float32),
            pltpu.VMEM((T, N_EMBD), jnp.float32),
            pltpu.VMEM((2, N_EMBD), jnp.float32),
            pltpu.VMEM((2, N_EMBD), jnp.float32),
            pltpu.SemaphoreType.DMA,
        ],
    )
    return kern(ids2, tt2, tok_table, pos_table, seg_table, gamma, beta)


def kernel(input_ids, token_type_ids, tok_table, pos_table, seg_table,
           ln_gamma, ln_beta):
    ids2 = jnp.asarray(input_ids, jnp.int32).reshape(ROWS // HALF, HALF)
    tt2 = jnp.asarray(token_type_ids, jnp.int32).reshape(ROWS)
    out = _bert_embed_sc(ids2, tt2, tok_table, pos_table, seg_table,
                         jnp.asarray(ln_gamma, jnp.float32),
                         jnp.asarray(ln_beta, jnp.float32))
    return out.reshape(B, T, N_EMBD)


# speedup vs baseline: 4.4507x; 1.3782x over previous
import jax
import jax.numpy as jnp
from jax import lax
from jax.experimental import pallas as pl
from jax.experimental.pallas import tpu as pltpu, tpu_sc as plsc

VOCAB = 100000
N_EMBD = 128
B, T = 1024, 200

NC, NS, L = 2, 16, 16
NW = NC * NS
ROWS = B * T
ROWS_W = ROWS // NW
SEQ_W = ROWS_W // T
HALF = T // 2
NV = N_EMBD // L

_EPS = 1e-5
_RSQRT_MAGIC = 0x5F3759DF


def _rsqrt16(x):
    i = plsc.bitcast(x, jnp.int32)
    y = plsc.bitcast(jnp.int32(_RSQRT_MAGIC) - (i >> 1), jnp.float32)
    for _ in range(2):
        y = y * (1.5 - 0.5 * x * y * y)
    return y


def _sc_body(ids_hbm, tt_hbm, tok_hbm, pos_hbm, seg_hbm, gam_hbm, bet_hbm,
             out_hbm, idx_v, ttv_v, buf_v, ps_v, seg_v, gb_v, gsem):
    wid = lax.axis_index("s") * NC + lax.axis_index("c")

    pltpu.sync_copy(ids_hbm.at[pl.ds(wid * (2 * SEQ_W), 2 * SEQ_W)], idx_v)
    pltpu.sync_copy(tt_hbm.at[pl.ds(wid * ROWS_W, ROWS_W)],
                    ttv_v.at[pl.ds(0, ROWS_W)])
    pltpu.sync_copy(pos_hbm.at[pl.ds(0, T)], ps_v.at[0])
    pltpu.sync_copy(pos_hbm.at[pl.ds(0, T)], ps_v.at[1])
    pltpu.sync_copy(seg_hbm, seg_v)
    pltpu.sync_copy(gam_hbm, gb_v.at[0])
    pltpu.sync_copy(bet_hbm, gb_v.at[1])

    seg0 = [seg_v[0, pl.ds(L * k, L)] for k in range(NV)]
    seg1 = [seg_v[1, pl.ds(L * k, L)] for k in range(NV)]
    gam = [gb_v[0, pl.ds(L * k, L)] for k in range(NV)]
    bet = [gb_v[1, pl.ds(L * k, L)] for k in range(NV)]

    def build_ps(r, _):
        for k in range(NV):
            ps_v[0, r, pl.ds(L * k, L)] = ps_v[0, r, pl.ds(L * k, L)] + seg0[k]
            ps_v[1, r, pl.ds(L * k, L)] = ps_v[1, r, pl.ds(L * k, L)] + seg1[k]
        return 0

    lax.fori_loop(0, T, build_ps, 0, unroll=2)

    def ln_row(j, tts):
        acc = [buf_v[j, pl.ds(L * k, L)] + ps_v[tts, j, pl.ds(L * k, L)]
               for k in range(NV)]
        s01 = (acc[0] + acc[1]) + (acc[2] + acc[3])
        s23 = (acc[4] + acc[5]) + (acc[6] + acc[7])
        s = s01 + s23
        sq = None
        for k in range(NV):
            d2 = acc[k] * acc[k]
            sq = d2 if sq is None else sq + d2
        mean = jnp.sum(s) * (1.0 / N_EMBD)
        ex2 = jnp.sum(sq) * (1.0 / N_EMBD)
        var = ex2 - mean * mean
        rstd = _rsqrt16(jnp.full((L,), var + _EPS, jnp.float32))
        for k in range(NV):
            a = rstd * gam[k]
            buf_v[j, pl.ds(L * k, L)] = acc[k] * a + (bet[k] - mean * a)

    def do_seq(c, _):
        cp0 = pltpu.async_copy(tok_hbm.at[idx_v.at[2 * c]],
                               buf_v.at[pl.ds(0, HALF)], gsem)
        cp1 = pltpu.async_copy(tok_hbm.at[idx_v.at[2 * c + 1]],
                               buf_v.at[pl.ds(HALF, HALF)], gsem)
        cp0.wait()
        cp1.wait()

        def do_pair(p, _):
            j = 2 * p
            ttpair = ttv_v[pl.ds(c * T + j, L)]
            ln_row(j, ttpair[0])
            ln_row(j + 1, ttpair[1])
            return 0

        lax.fori_loop(0, T // 2, do_pair, 0)
        pltpu.sync_copy(buf_v, out_hbm.at[pl.ds((wid * SEQ_W + c) * T, T)])
        return 0

    lax.fori_loop(0, SEQ_W, do_seq, 0)


@jax.jit
def _bert_embed_sc(ids2, tt2, tok_table, pos_table, seg_table, gamma, beta):
    kern = pl.kernel(
        _sc_body,
        out_type=jax.ShapeDtypeStruct((ROWS, N_EMBD), jnp.float32),
        mesh=plsc.VectorSubcoreMesh(core_axis_name="c", subcore_axis_name="s"),
        compiler_params=pltpu.CompilerParams(needs_layout_passes=False),
        scratch_types=[
            pltpu.VMEM((2 * SEQ_W, HALF), jnp.int32),
            pltpu.VMEM((ROWS_W + L,), jnp.int32),
            pltpu.VMEM((T, N_EMBD), jnp.float32),
            pltpu.VMEM((2, T, N_EMBD), jnp.float32),
            pltpu.VMEM((2, N_EMBD), jnp.float32),
            pltpu.VMEM((2, N_EMBD), jnp.float32),
            pltpu.SemaphoreType.DMA,
        ],
    )
    return kern(ids2, tt2, tok_table, pos_table, seg_table, gamma, beta)


def kernel(input_ids, token_type_ids, tok_table, pos_table, seg_table,
           ln_gamma, ln_beta):
    ids2 = jnp.asarray(input_ids, jnp.int32).reshape(ROWS // HALF, HALF)
    tt2 = jnp.asarray(token_type_ids, jnp.int32).reshape(ROWS)
    out = _bert_embed_sc(ids2, tt2, tok_table, pos_table, seg_table,
                         jnp.asarray(ln_gamma, jnp.float32),
                         jnp.asarray(ln_beta, jnp.float32))
    return out.reshape(B, T, N_EMBD)
